# Initial kernel scaffold; baseline (speedup 1.0000x reference)
import jax
import jax.numpy as jnp
from jax.experimental import pallas as pl


def kernel(x, weights, mask_token):
    raise NotImplementedError("write your pallas kernel here")


# speedup vs baseline: 2.9170x; 2.9170x over previous
import jax
import jax.numpy as jnp
from jax import lax
from jax.experimental import pallas as pl
from jax.experimental.pallas import tpu as pltpu

_PATCH_LEN = 32
_NEG_HUGE = -3.0e38


def _compute_mask(w):
    p, f = w.shape
    iota = lax.broadcasted_iota(jnp.int32, (p, f), 1)

    v = w
    sel = jnp.zeros_like(w, jnp.bool_)
    for _ in range(_PATCH_LEN):
        cur = jnp.max(v, axis=1, keepdims=True)
        eq = v == cur
        first = jnp.min(jnp.where(eq, iota, f), axis=1, keepdims=True)
        hit = iota == first
        v = jnp.where(hit, _NEG_HUGE, v)
        sel = jnp.logical_or(sel, hit)
    return sel.astype(jnp.float32)


def _patch_kernel(x_ref, w_ref, mt_ref, out_ref, m_ref):
    @pl.when(pl.program_id(0) == 0)
    def _():
        m_ref[...] = _compute_mask(w_ref[...])

    m = m_ref[...]
    xb = x_ref[...]
    bb = xb.shape[0]
    sel = m[None] != 0
    out_ref[:, :, : m.shape[1]] = jnp.where(
        sel, xb[:, None, :], mt_ref[0][None, None, :]
    )
    out_ref[:, :, m.shape[1]:] = jnp.broadcast_to(m[None], (bb,) + m.shape)


def kernel(x, weights, mask_token):
    batch, n_features = x.shape
    n_patches = weights.shape[0]
    b_blk = 32
    grid = (batch // b_blk,)

    return pl.pallas_call(
        _patch_kernel,
        grid=grid,
        in_specs=[
            pl.BlockSpec((b_blk, n_features), lambda i: (i, 0)),
            pl.BlockSpec((n_patches, n_features), lambda i: (0, 0)),
            pl.BlockSpec((1, n_features), lambda i: (0, 0)),
        ],
        out_specs=pl.BlockSpec((b_blk, n_patches, 2 * n_features), lambda i: (i, 0, 0)),
        out_shape=jax.ShapeDtypeStruct((batch, n_patches, 2 * n_features), jnp.float32),
        scratch_shapes=[pltpu.VMEM((n_patches, n_features), jnp.float32)],
    )(x, weights, mask_token.reshape(1, n_features))
